# Initial kernel scaffold; baseline (speedup 1.0000x reference)
import jax
import jax.numpy as jnp
from jax.experimental import pallas as pl


def kernel(inputs, W):
    raise NotImplementedError("write your pallas kernel here")


# speedup vs baseline: 2.4072x; 2.4072x over previous
import jax
import jax.numpy as jnp
from jax import lax
from jax.experimental import pallas as pl
from jax.experimental.pallas import tpu as pltpu

_NUM_EMB = 1024
_EMB_DIM = 64
_COMMIT = 0.25
_TBLK = 1024


def _vq_body(flat_ref, w_ref, wsq_ref, enc_ref, qst_ref, loss_ref, ppl_ref,
             sse_ref, cnt_ref):
    i = pl.program_id(0)
    nsteps = pl.num_programs(0)
    xb = flat_ref[...]
    w = w_ref[...]
    xsq = jnp.sum(xb * xb, axis=1, keepdims=True)
    m = lax.dot_general(xb, w, (((1,), (1,)), ((), ())),
                        preferred_element_type=jnp.float32)
    dist = (xsq + wsq_ref[...]) - 2.0 * m
    dmin = jnp.min(dist, axis=1, keepdims=True)
    iota = lax.broadcasted_iota(jnp.int32, dist.shape, 1)
    idx = jnp.min(jnp.where(dist == dmin, iota, _NUM_EMB),
                  axis=1, keepdims=True)
    enc = (iota == idx).astype(jnp.float32)
    enc_ref[...] = enc
    q = lax.dot_general(enc, w, (((1,), (0,)), ((), ())),
                        preferred_element_type=jnp.float32,
                        precision=lax.Precision.HIGHEST)
    d = q - xb
    qst_ref[...] = xb + d
    sse_part = jnp.sum(d * d)
    cnt_part = jnp.sum(enc, axis=0, keepdims=True)

    @pl.when(i == 0)
    def _init():
        sse_ref[0] = sse_part
        cnt_ref[...] = cnt_part

    @pl.when(i != 0)
    def _acc():
        sse_ref[0] += sse_part
        cnt_ref[...] += cnt_part

    @pl.when(i == nsteps - 1)
    def _fin():
        n_tok = nsteps * _TBLK
        mean = sse_ref[0] / (n_tok * _EMB_DIM)
        loss_ref[...] = jnp.reshape(mean + _COMMIT * mean, (1, 1))
        avg = cnt_ref[...] / n_tok
        ent = jnp.sum(avg * jnp.log(avg + 1e-10), axis=1, keepdims=True)
        ppl_ref[...] = jnp.exp(-ent)


def kernel(inputs, W):
    B, C, H, Wd = inputs.shape
    x = jnp.transpose(inputs, (0, 2, 3, 1))
    flat = x.reshape(-1, C)
    N = flat.shape[0]
    wsq = jnp.sum(W ** 2, axis=1).reshape(1, _NUM_EMB)
    grid = N // _TBLK

    enc, qst, loss, ppl = pl.pallas_call(
        _vq_body,
        grid=(grid,),
        in_specs=[
            pl.BlockSpec((_TBLK, C), lambda i: (i, 0)),
            pl.BlockSpec((_NUM_EMB, C), lambda i: (0, 0)),
            pl.BlockSpec((1, _NUM_EMB), lambda i: (0, 0)),
        ],
        out_specs=[
            pl.BlockSpec((_TBLK, _NUM_EMB), lambda i: (i, 0)),
            pl.BlockSpec((_TBLK, C), lambda i: (i, 0)),
            pl.BlockSpec((1, 1), lambda i: (0, 0)),
            pl.BlockSpec((1, 1), lambda i: (0, 0)),
        ],
        out_shape=[
            jax.ShapeDtypeStruct((N, _NUM_EMB), jnp.float32),
            jax.ShapeDtypeStruct((N, C), jnp.float32),
            jax.ShapeDtypeStruct((1, 1), jnp.float32),
            jax.ShapeDtypeStruct((1, 1), jnp.float32),
        ],
        scratch_shapes=[
            pltpu.SMEM((1,), jnp.float32),
            pltpu.VMEM((1, _NUM_EMB), jnp.float32),
        ],
    )(flat, W, wsq)

    quantized_st = jnp.transpose(qst.reshape(B, H, Wd, C), (0, 3, 1, 2))
    return (loss[0, 0], quantized_st, ppl[0, 0], enc)


# speedup vs baseline: 4.3287x; 1.7982x over previous
import jax
import jax.numpy as jnp
from jax import lax
from jax.experimental import pallas as pl
from jax.experimental.pallas import tpu as pltpu

_NUM_EMB = 1024
_EMB_DIM = 64
_COMMIT = 0.25
_TBLK = 1024


def _vq_body(flat_ref, w_ref, wsq_ref, enc_ref, qst_ref, loss_ref, ppl_ref,
             sse_ref, cnt_ref):
    i = pl.program_id(0)
    nsteps = pl.num_programs(0)
    xb = flat_ref[...]
    w = w_ref[...]
    xsq = jnp.sum(xb * xb, axis=1, keepdims=True)
    m = lax.dot_general(xb, w, (((1,), (1,)), ((), ())),
                        preferred_element_type=jnp.float32)
    dist = (xsq + wsq_ref[...]) - 2.0 * m
    dmin = jnp.min(dist, axis=1, keepdims=True)
    iota = lax.broadcasted_iota(jnp.int32, dist.shape, 1)
    idx = jnp.min(jnp.where(dist == dmin, iota, _NUM_EMB),
                  axis=1, keepdims=True)
    enc = (iota == idx).astype(jnp.float32)
    enc_ref[...] = enc
    q = lax.dot_general(enc, w, (((1,), (0,)), ((), ())),
                        preferred_element_type=jnp.float32)
    d = q - xb
    qst_ref[...] = xb + d
    sse_part = jnp.sum(d * d)
    cnt_part = jnp.sum(enc, axis=0, keepdims=True)

    @pl.when(i == 0)
    def _init():
        sse_ref[0] = sse_part
        cnt_ref[...] = cnt_part

    @pl.when(i != 0)
    def _acc():
        sse_ref[0] += sse_part
        cnt_ref[...] += cnt_part

    @pl.when(i == nsteps - 1)
    def _fin():
        n_tok = nsteps * _TBLK
        mean = sse_ref[0] / (n_tok * _EMB_DIM)
        loss_ref[...] = jnp.reshape(mean + _COMMIT * mean, (1, 1))
        avg = cnt_ref[...] / n_tok
        ent = jnp.sum(avg * jnp.log(avg + 1e-10), axis=1, keepdims=True)
        ppl_ref[...] = jnp.exp(-ent)


def kernel(inputs, W):
    B, C, H, Wd = inputs.shape
    x = jnp.transpose(inputs, (0, 2, 3, 1))
    flat = x.reshape(-1, C)
    N = flat.shape[0]
    wsq = jnp.sum(W ** 2, axis=1).reshape(1, _NUM_EMB)
    grid = N // _TBLK

    enc, qst, loss, ppl = pl.pallas_call(
        _vq_body,
        grid=(grid,),
        in_specs=[
            pl.BlockSpec((_TBLK, C), lambda i: (i, 0)),
            pl.BlockSpec((_NUM_EMB, C), lambda i: (0, 0)),
            pl.BlockSpec((1, _NUM_EMB), lambda i: (0, 0)),
        ],
        out_specs=[
            pl.BlockSpec((_TBLK, _NUM_EMB), lambda i: (i, 0)),
            pl.BlockSpec((_TBLK, C), lambda i: (i, 0)),
            pl.BlockSpec((1, 1), lambda i: (0, 0)),
            pl.BlockSpec((1, 1), lambda i: (0, 0)),
        ],
        out_shape=[
            jax.ShapeDtypeStruct((N, _NUM_EMB), jnp.float32),
            jax.ShapeDtypeStruct((N, C), jnp.float32),
            jax.ShapeDtypeStruct((1, 1), jnp.float32),
            jax.ShapeDtypeStruct((1, 1), jnp.float32),
        ],
        scratch_shapes=[
            pltpu.SMEM((1,), jnp.float32),
            pltpu.VMEM((1, _NUM_EMB), jnp.float32),
        ],
    )(flat, W, wsq)

    quantized_st = jnp.transpose(qst.reshape(B, H, Wd, C), (0, 3, 1, 2))
    return (loss[0, 0], quantized_st, ppl[0, 0], enc)


# speedup vs baseline: 4.5661x; 1.0548x over previous
import jax
import jax.numpy as jnp
from jax import lax
from jax.experimental import pallas as pl
from jax.experimental.pallas import tpu as pltpu

_NUM_EMB = 1024
_EMB_DIM = 64
_COMMIT = 0.25
_TBLK = 1024


def _vq_body(flat_ref, w_ref, w2_ref, wsq_ref, iota_ref, enc_ref, qst_ref,
             loss_ref, ppl_ref, sse_ref, cnt_ref):
    i = pl.program_id(0)
    nsteps = pl.num_programs(0)
    xb = flat_ref[...]
    w = w_ref[...]
    xsq = jnp.sum(xb * xb, axis=1, keepdims=True)
    m2 = lax.dot_general(xb, w2_ref[...], (((1,), (1,)), ((), ())),
                         preferred_element_type=jnp.float32)
    dist = (xsq + wsq_ref[...]) - m2
    dmin = jnp.min(dist, axis=1, keepdims=True)
    iota = iota_ref[...]
    idx = jnp.min(jnp.where(dist == dmin, iota, float(_NUM_EMB)),
                  axis=1, keepdims=True)
    enc = (iota == idx).astype(jnp.float32)
    enc_ref[...] = enc
    q = lax.dot_general(enc, w, (((1,), (0,)), ((), ())),
                        preferred_element_type=jnp.float32)
    d = q - xb
    qst_ref[...] = xb + d
    sse_part = jnp.sum(d * d)
    cnt_part = jnp.sum(enc, axis=0, keepdims=True)

    @pl.when(i == 0)
    def _init():
        sse_ref[0] = sse_part
        cnt_ref[...] = cnt_part

    @pl.when(i != 0)
    def _acc():
        sse_ref[0] += sse_part
        cnt_ref[...] += cnt_part

    @pl.when(i == nsteps - 1)
    def _fin():
        n_tok = nsteps * _TBLK
        mean = sse_ref[0] / (n_tok * _EMB_DIM)
        loss_ref[...] = jnp.reshape(mean + _COMMIT * mean, (1, 1))
        avg = cnt_ref[...] / n_tok
        ent = jnp.sum(avg * jnp.log(avg + 1e-10), axis=1, keepdims=True)
        ppl_ref[...] = jnp.exp(-ent)


def kernel(inputs, W):
    B, C, H, Wd = inputs.shape
    x = jnp.transpose(inputs, (0, 2, 3, 1))
    flat = x.reshape(-1, C)
    N = flat.shape[0]
    wsq = jnp.sum(W ** 2, axis=1).reshape(1, _NUM_EMB)
    w2 = W + W
    iota = lax.broadcasted_iota(jnp.float32, (1, _NUM_EMB), 1)
    grid = N // _TBLK

    enc, qst, loss, ppl = pl.pallas_call(
        _vq_body,
        grid=(grid,),
        in_specs=[
            pl.BlockSpec((_TBLK, C), lambda i: (i, 0)),
            pl.BlockSpec((_NUM_EMB, C), lambda i: (0, 0)),
            pl.BlockSpec((_NUM_EMB, C), lambda i: (0, 0)),
            pl.BlockSpec((1, _NUM_EMB), lambda i: (0, 0)),
            pl.BlockSpec((1, _NUM_EMB), lambda i: (0, 0)),
        ],
        out_specs=[
            pl.BlockSpec((_TBLK, _NUM_EMB), lambda i: (i, 0)),
            pl.BlockSpec((_TBLK, C), lambda i: (i, 0)),
            pl.BlockSpec((1, 1), lambda i: (0, 0)),
            pl.BlockSpec((1, 1), lambda i: (0, 0)),
        ],
        out_shape=[
            jax.ShapeDtypeStruct((N, _NUM_EMB), jnp.float32),
            jax.ShapeDtypeStruct((N, C), jnp.float32),
            jax.ShapeDtypeStruct((1, 1), jnp.float32),
            jax.ShapeDtypeStruct((1, 1), jnp.float32),
        ],
        scratch_shapes=[
            pltpu.SMEM((1,), jnp.float32),
            pltpu.VMEM((1, _NUM_EMB), jnp.float32),
        ],
    )(flat, W, w2, wsq, iota)

    quantized_st = jnp.transpose(qst.reshape(B, H, Wd, C), (0, 3, 1, 2))
    return (loss[0, 0], quantized_st, ppl[0, 0], enc)


# speedup vs baseline: 5.0121x; 1.0977x over previous
import jax
import jax.numpy as jnp
from jax import lax
from jax.experimental import pallas as pl
from jax.experimental.pallas import tpu as pltpu

_NUM_EMB = 1024
_EMB_DIM = 64
_COMMIT = 0.25
_TBLK = 1024


def _vq_body(flat_ref, w_ref, w2_ref, wsq_ref, iota_ref, enc_ref, qst_ref,
             loss_ref, ppl_ref, sse_ref, cnt_ref):
    i = pl.program_id(0)
    nsteps = pl.num_programs(0)
    xb = flat_ref[...]
    w = w_ref[...]
    xsq = jnp.sum(xb * xb, axis=1, keepdims=True)
    m2 = lax.dot_general(xb, w2_ref[...], (((1,), (1,)), ((), ())),
                         preferred_element_type=jnp.float32)
    dist = (xsq + wsq_ref[...]) - m2
    dmin = jnp.min(dist, axis=1, keepdims=True)
    iota = iota_ref[...]
    idx = jnp.min(jnp.where(dist == dmin, iota, float(_NUM_EMB)),
                  axis=1, keepdims=True)
    enc = (iota == idx).astype(jnp.float32)
    enc_ref[...] = enc
    q = lax.dot_general(enc, w, (((1,), (0,)), ((), ())),
                        preferred_element_type=jnp.float32)
    d = q - xb
    qst_ref[...] = xb + d
    sse_part = jnp.sum(d * d)
    ones_row = jnp.full((1, xb.shape[0]), 1.0, jnp.float32)
    cnt_part = lax.dot_general(ones_row, enc, (((1,), (0,)), ((), ())),
                               preferred_element_type=jnp.float32)

    @pl.when(i == 0)
    def _init():
        sse_ref[0] = sse_part
        cnt_ref[...] = cnt_part

    @pl.when(i != 0)
    def _acc():
        sse_ref[0] += sse_part
        cnt_ref[...] += cnt_part

    @pl.when(i == nsteps - 1)
    def _fin():
        n_tok = nsteps * _TBLK
        mean = sse_ref[0] / (n_tok * _EMB_DIM)
        loss_ref[...] = jnp.reshape(mean + _COMMIT * mean, (1, 1))
        avg = cnt_ref[...] / n_tok
        ent = jnp.sum(avg * jnp.log(avg + 1e-10), axis=1, keepdims=True)
        ppl_ref[...] = jnp.exp(-ent)


def kernel(inputs, W):
    B, C, H, Wd = inputs.shape
    x = jnp.transpose(inputs, (0, 2, 3, 1))
    flat = x.reshape(-1, C)
    N = flat.shape[0]
    wsq = jnp.sum(W ** 2, axis=1).reshape(1, _NUM_EMB)
    w2 = W + W
    iota = lax.broadcasted_iota(jnp.float32, (1, _NUM_EMB), 1)
    grid = N // _TBLK

    enc, qst, loss, ppl = pl.pallas_call(
        _vq_body,
        grid=(grid,),
        in_specs=[
            pl.BlockSpec((_TBLK, C), lambda i: (i, 0)),
            pl.BlockSpec((_NUM_EMB, C), lambda i: (0, 0)),
            pl.BlockSpec((_NUM_EMB, C), lambda i: (0, 0)),
            pl.BlockSpec((1, _NUM_EMB), lambda i: (0, 0)),
            pl.BlockSpec((1, _NUM_EMB), lambda i: (0, 0)),
        ],
        out_specs=[
            pl.BlockSpec((_TBLK, _NUM_EMB), lambda i: (i, 0)),
            pl.BlockSpec((_TBLK, C), lambda i: (i, 0)),
            pl.BlockSpec((1, 1), lambda i: (0, 0)),
            pl.BlockSpec((1, 1), lambda i: (0, 0)),
        ],
        out_shape=[
            jax.ShapeDtypeStruct((N, _NUM_EMB), jnp.float32),
            jax.ShapeDtypeStruct((N, C), jnp.float32),
            jax.ShapeDtypeStruct((1, 1), jnp.float32),
            jax.ShapeDtypeStruct((1, 1), jnp.float32),
        ],
        scratch_shapes=[
            pltpu.SMEM((1,), jnp.float32),
            pltpu.VMEM((1, _NUM_EMB), jnp.float32),
        ],
    )(flat, W, w2, wsq, iota)

    quantized_st = jnp.transpose(qst.reshape(B, H, Wd, C), (0, 3, 1, 2))
    return (loss[0, 0], quantized_st, ppl[0, 0], enc)


# speedup vs baseline: 5.3234x; 1.0621x over previous
import jax
import jax.numpy as jnp
from jax import lax
from jax.experimental import pallas as pl
from jax.experimental.pallas import tpu as pltpu

_NUM_EMB = 1024
_EMB_DIM = 64
_COMMIT = 0.25
_TBLK = 2048


def _vq_body(flat_ref, w_ref, w2_ref, wsq_ref, iota_ref, enc_ref, qst_ref,
             loss_ref, ppl_ref, sse_ref, cnt_ref):
    i = pl.program_id(0)
    nsteps = pl.num_programs(0)
    xb = flat_ref[...]
    w = w_ref[...]
    xsq = jnp.sum(xb * xb, axis=1, keepdims=True)
    m2 = lax.dot_general(xb, w2_ref[...], (((1,), (1,)), ((), ())),
                         preferred_element_type=jnp.float32)
    dist = (xsq + wsq_ref[...]) - m2
    dmin = jnp.min(dist, axis=1, keepdims=True)
    iota = iota_ref[...]
    idx = jnp.min(jnp.where(dist == dmin, iota, float(_NUM_EMB)),
                  axis=1, keepdims=True)
    enc = (iota == idx).astype(jnp.float32)
    enc_ref[...] = enc
    q = lax.dot_general(enc, w, (((1,), (0,)), ((), ())),
                        preferred_element_type=jnp.float32)
    d = q - xb
    qst_ref[...] = xb + d
    sse_part = jnp.sum(d * d)
    ones_row = jnp.full((1, xb.shape[0]), 1.0, jnp.float32)
    cnt_part = lax.dot_general(ones_row, enc, (((1,), (0,)), ((), ())),
                               preferred_element_type=jnp.float32)

    @pl.when(i == 0)
    def _init():
        sse_ref[0] = sse_part
        cnt_ref[...] = cnt_part

    @pl.when(i != 0)
    def _acc():
        sse_ref[0] += sse_part
        cnt_ref[...] += cnt_part

    @pl.when(i == nsteps - 1)
    def _fin():
        n_tok = nsteps * _TBLK
        mean = sse_ref[0] / (n_tok * _EMB_DIM)
        loss_ref[...] = jnp.reshape(mean + _COMMIT * mean, (1, 1))
        avg = cnt_ref[...] / n_tok
        ent = jnp.sum(avg * jnp.log(avg + 1e-10), axis=1, keepdims=True)
        ppl_ref[...] = jnp.exp(-ent)


def kernel(inputs, W):
    B, C, H, Wd = inputs.shape
    x = jnp.transpose(inputs, (0, 2, 3, 1))
    flat = x.reshape(-1, C)
    N = flat.shape[0]
    wsq = jnp.sum(W ** 2, axis=1).reshape(1, _NUM_EMB)
    w2 = W + W
    iota = lax.broadcasted_iota(jnp.float32, (1, _NUM_EMB), 1)
    grid = N // _TBLK

    enc, qst, loss, ppl = pl.pallas_call(
        _vq_body,
        grid=(grid,),
        in_specs=[
            pl.BlockSpec((_TBLK, C), lambda i: (i, 0)),
            pl.BlockSpec((_NUM_EMB, C), lambda i: (0, 0)),
            pl.BlockSpec((_NUM_EMB, C), lambda i: (0, 0)),
            pl.BlockSpec((1, _NUM_EMB), lambda i: (0, 0)),
            pl.BlockSpec((1, _NUM_EMB), lambda i: (0, 0)),
        ],
        out_specs=[
            pl.BlockSpec((_TBLK, _NUM_EMB), lambda i: (i, 0)),
            pl.BlockSpec((_TBLK, C), lambda i: (i, 0)),
            pl.BlockSpec((1, 1), lambda i: (0, 0)),
            pl.BlockSpec((1, 1), lambda i: (0, 0)),
        ],
        out_shape=[
            jax.ShapeDtypeStruct((N, _NUM_EMB), jnp.float32),
            jax.ShapeDtypeStruct((N, C), jnp.float32),
            jax.ShapeDtypeStruct((1, 1), jnp.float32),
            jax.ShapeDtypeStruct((1, 1), jnp.float32),
        ],
        scratch_shapes=[
            pltpu.SMEM((1,), jnp.float32),
            pltpu.VMEM((1, _NUM_EMB), jnp.float32),
        ],
    )(flat, W, w2, wsq, iota)

    quantized_st = jnp.transpose(qst.reshape(B, H, Wd, C), (0, 3, 1, 2))
    return (loss[0, 0], quantized_st, ppl[0, 0], enc)


# speedup vs baseline: 5.4602x; 1.0257x over previous
import jax
import jax.numpy as jnp
from jax import lax
from jax.experimental import pallas as pl
from jax.experimental.pallas import tpu as pltpu

_NUM_EMB = 1024
_EMB_DIM = 64
_COMMIT = 0.25
_TBLK = 4096


def _vq_body(flat_ref, w_ref, w2_ref, wsq_ref, iota_ref, enc_ref, qst_ref,
             loss_ref, ppl_ref, sse_ref, cnt_ref):
    i = pl.program_id(0)
    nsteps = pl.num_programs(0)
    xb = flat_ref[...]
    w = w_ref[...]
    xsq = jnp.sum(xb * xb, axis=1, keepdims=True)
    m2 = lax.dot_general(xb, w2_ref[...], (((1,), (1,)), ((), ())),
                         preferred_element_type=jnp.float32)
    dist = (xsq + wsq_ref[...]) - m2
    dmin = jnp.min(dist, axis=1, keepdims=True)
    iota = iota_ref[...]
    idx = jnp.min(jnp.where(dist == dmin, iota, float(_NUM_EMB)),
                  axis=1, keepdims=True)
    enc = (iota == idx).astype(jnp.float32)
    enc_ref[...] = enc
    q = lax.dot_general(enc, w, (((1,), (0,)), ((), ())),
                        preferred_element_type=jnp.float32)
    d = q - xb
    qst_ref[...] = xb + d
    sse_part = jnp.sum(d * d)
    ones_row = jnp.full((1, xb.shape[0]), 1.0, jnp.float32)
    cnt_part = lax.dot_general(ones_row, enc, (((1,), (0,)), ((), ())),
                               preferred_element_type=jnp.float32)

    @pl.when(i == 0)
    def _init():
        sse_ref[0] = sse_part
        cnt_ref[...] = cnt_part

    @pl.when(i != 0)
    def _acc():
        sse_ref[0] += sse_part
        cnt_ref[...] += cnt_part

    @pl.when(i == nsteps - 1)
    def _fin():
        n_tok = nsteps * _TBLK
        mean = sse_ref[0] / (n_tok * _EMB_DIM)
        loss_ref[...] = jnp.reshape(mean + _COMMIT * mean, (1, 1))
        avg = cnt_ref[...] / n_tok
        ent = jnp.sum(avg * jnp.log(avg + 1e-10), axis=1, keepdims=True)
        ppl_ref[...] = jnp.exp(-ent)


def kernel(inputs, W):
    B, C, H, Wd = inputs.shape
    x = jnp.transpose(inputs, (0, 2, 3, 1))
    flat = x.reshape(-1, C)
    N = flat.shape[0]
    wsq = jnp.sum(W ** 2, axis=1).reshape(1, _NUM_EMB)
    w2 = W + W
    iota = lax.broadcasted_iota(jnp.float32, (1, _NUM_EMB), 1)
    grid = N // _TBLK

    enc, qst, loss, ppl = pl.pallas_call(
        _vq_body,
        grid=(grid,),
        in_specs=[
            pl.BlockSpec((_TBLK, C), lambda i: (i, 0)),
            pl.BlockSpec((_NUM_EMB, C), lambda i: (0, 0)),
            pl.BlockSpec((_NUM_EMB, C), lambda i: (0, 0)),
            pl.BlockSpec((1, _NUM_EMB), lambda i: (0, 0)),
            pl.BlockSpec((1, _NUM_EMB), lambda i: (0, 0)),
        ],
        out_specs=[
            pl.BlockSpec((_TBLK, _NUM_EMB), lambda i: (i, 0)),
            pl.BlockSpec((_TBLK, C), lambda i: (i, 0)),
            pl.BlockSpec((1, 1), lambda i: (0, 0)),
            pl.BlockSpec((1, 1), lambda i: (0, 0)),
        ],
        out_shape=[
            jax.ShapeDtypeStruct((N, _NUM_EMB), jnp.float32),
            jax.ShapeDtypeStruct((N, C), jnp.float32),
            jax.ShapeDtypeStruct((1, 1), jnp.float32),
            jax.ShapeDtypeStruct((1, 1), jnp.float32),
        ],
        scratch_shapes=[
            pltpu.SMEM((1,), jnp.float32),
            pltpu.VMEM((1, _NUM_EMB), jnp.float32),
        ],
    )(flat, W, w2, wsq, iota)

    quantized_st = jnp.transpose(qst.reshape(B, H, Wd, C), (0, 3, 1, 2))
    return (loss[0, 0], quantized_st, ppl[0, 0], enc)
